# baseline (speedup 1.0000x reference)
import functools

import jax
import jax.numpy as jnp
from jax import lax
from jax.experimental import pallas as pl
from jax.experimental.pallas import tpu as pltpu
from jax.experimental.pallas import tpu_sc as plsc

_PATCH = 16
_ROWS = 20
_N = 1024
_L = 16
_NCHUNK = _N // _L
_KEY_BITS = 23
_BUCKET_SHIFT = 15
_LOW_MASK = (1 << _BUCKET_SHIFT) - 1
_NBUCKET = 1 << (_KEY_BITS - _BUCKET_SHIFT)
_SENTINEL = 0x7FFFFFFF


def _sc_body(ratio_hbm, rand_hbm, out_hbm, row_v, m_v, ratio_v, hist_v,
             comp_v, out_v, sem_row, sem_ratio):
    wid = lax.axis_index("s") * 2 + lax.axis_index("c")

    @pl.when(wid < _ROWS)
    def _():
        cp_row = pltpu.make_async_copy(rand_hbm.at[wid], row_v, sem_row)
        cp_row.start()
        cp_ratio = pltpu.make_async_copy(ratio_hbm, ratio_v, sem_ratio)
        cp_ratio.start()
        lane = lax.iota(jnp.int32, _L)
        ones = jnp.ones((_L,), jnp.int32)
        zeros = jnp.zeros((_L,), jnp.int32)

        def zero_body(i, carry):
            hist_v[pl.ds(i * _L, _L)] = zeros
            return carry

        lax.fori_loop(0, _NBUCKET // _L, zero_body, 0)
        cp_row.wait()
        cp_ratio.wait()

        def pass_a(i, carry):
            x = row_v[pl.ds(i * _L, _L)]
            m = (x * float(1 << _KEY_BITS)).astype(jnp.int32)
            m_v[pl.ds(i * _L, _L)] = m
            plsc.addupdate_scatter(hist_v, [m >> _BUCKET_SHIFT], ones)
            return carry

        lax.fori_loop(0, _NCHUNK, pass_a, 0, unroll=4)

        ratio = ratio_v[...]
        k_vec = (ratio * float(_N)).astype(jnp.int32)

        def scan_hist(i, carry):
            run, b_acc, cb_acc = carry
            h = hist_v[pl.ds(i * _L, _L)]
            cum = plsc.cumsum(h) + run
            lt = cum < k_vec
            return (jnp.broadcast_to(jnp.max(cum), (_L,)),
                    b_acc + jnp.where(lt, 1, 0),
                    cb_acc + jnp.where(lt, h, 0))

        _, b_acc, cb_acc = lax.fori_loop(0, _NBUCKET // _L, scan_hist,
                                         (zeros, zeros, zeros), unroll=2)
        b_vec = jnp.broadcast_to(jnp.sum(b_acc), (_L,))
        cb_vec = jnp.broadcast_to(jnp.sum(cb_acc), (_L,))
        rrem = k_vec - cb_vec

        comp_v[...] = jnp.full((_L,), _SENTINEL, jnp.int32)

        def pass_b(i, off):
            m = m_v[pl.ds(i * _L, _L)]
            inb = (m >> _BUCKET_SHIFT) == b_vec
            pc = plsc.cumsum(jnp.where(inb, 1, 0))
            comb = ((m & _LOW_MASK) << 10) | (lane + i * _L)
            plsc.store_scatter(comp_v, [off + pc - 1], comb, mask=inb)
            return off + plsc.all_reduce_population_count(inb)

        lax.fori_loop(0, _NCHUNK, pass_b, zeros, unroll=2)
        srt = jnp.sort(comp_v[...])
        sel = jnp.clip(rrem - 1, 0, _L - 1)
        tcomb = jnp.broadcast_to(jnp.sum(jnp.where(lane == sel, srt, 0)),
                                 (_L,))
        tm = (b_vec << _BUCKET_SHIFT) | (tcomb >> 10)
        tidx = tcomb & (_N - 1)
        valid = k_vec > 0

        def pass_c(i, carry):
            m = m_v[pl.ds(i * _L, _L)]
            gi = lane + i * _L
            selm = (m < tm) | ((m == tm) & (gi <= tidx))
            out_v[pl.ds(i * _L, _L)] = jnp.where(selm & valid, 1, 0)
            return carry

        lax.fori_loop(0, _NCHUNK, pass_c, 0, unroll=4)
        pltpu.sync_copy(out_v, out_hbm.at[wid])


@functools.partial(jax.jit, static_argnums=())
def _run_sc(ratio_rep, rand):
    mesh = plsc.VectorSubcoreMesh(core_axis_name="c", subcore_axis_name="s")
    fn = pl.kernel(
        _sc_body,
        out_type=jax.ShapeDtypeStruct((_ROWS, _N), jnp.int32),
        mesh=mesh,
        scratch_types=[
            pltpu.VMEM((_N,), jnp.float32),
            pltpu.VMEM((_N,), jnp.int32),
            pltpu.VMEM((_L,), jnp.float32),
            pltpu.VMEM((_NBUCKET,), jnp.int32),
            pltpu.VMEM((_L,), jnp.int32),
            pltpu.VMEM((_N,), jnp.int32),
            pltpu.SemaphoreType.DMA,
            pltpu.SemaphoreType.DMA,
        ],
        compiler_params=pltpu.CompilerParams(needs_layout_passes=False),
    )
    return fn(ratio_rep, rand)


def kernel(mask_frame, mask_ratio):
    B, T, C, H, W = mask_frame.shape
    h = H // _PATCH
    w = W // _PATCH
    rand = jax.random.uniform(jax.random.key(42), (_ROWS, _N),
                              dtype=jnp.float32)
    ratio_rep = jnp.broadcast_to(mask_ratio.astype(jnp.float32), (_L,))
    out = _run_sc(ratio_rep, rand)
    return out.astype(jnp.bool_).reshape(B, T, h, w)


# speedup vs baseline: 1.0322x; 1.0322x over previous
import functools

import jax
import jax.numpy as jnp
from jax import lax
from jax.experimental import pallas as pl
from jax.experimental.pallas import tpu as pltpu
from jax.experimental.pallas import tpu_sc as plsc

_PATCH = 16
_ROWS = 20
_N = 1024
_L = 16
_NCHUNK = _N // _L
_KEY_BITS = 23
_BUCKET_SHIFT = 15
_LOW_MASK = (1 << _BUCKET_SHIFT) - 1
_NBUCKET = 1 << (_KEY_BITS - _BUCKET_SHIFT)
_SENTINEL = 0x7FFFFFFF


def _sc_body(ratio_hbm, rand_hbm, out_hbm, row_v, m_v, ratio_v, hist_v,
             comp_v, out_v, sem_row, sem_ratio):
    wid = lax.axis_index("s") * 2 + lax.axis_index("c")

    @pl.when(wid < _ROWS)
    def _():
        cp_row = pltpu.make_async_copy(rand_hbm.at[wid], row_v, sem_row)
        cp_row.start()
        cp_ratio = pltpu.make_async_copy(ratio_hbm, ratio_v, sem_ratio)
        cp_ratio.start()
        lane = lax.iota(jnp.int32, _L)
        ones = jnp.ones((_L,), jnp.int32)
        zeros = jnp.zeros((_L,), jnp.int32)

        def zero_body(i, carry):
            hist_v[pl.ds(i * _L, _L)] = zeros
            return carry

        lax.fori_loop(0, _NBUCKET // _L, zero_body, 0)
        cp_row.wait()
        cp_ratio.wait()

        def pass_a(i, carry):
            x = row_v[pl.ds(i * _L, _L)]
            m = (x * float(1 << _KEY_BITS)).astype(jnp.int32)
            m_v[pl.ds(i * _L, _L)] = m
            plsc.addupdate_scatter(hist_v, [m >> _BUCKET_SHIFT], ones)
            return carry

        lax.fori_loop(0, _NCHUNK, pass_a, 0, unroll=2)

        ratio = ratio_v[...]
        k_vec = (ratio * float(_N)).astype(jnp.int32)

        def scan_hist(i, carry):
            run, b_acc, cb_acc = carry
            h = hist_v[pl.ds(i * _L, _L)]
            cum = plsc.cumsum(h) + run
            lt = cum < k_vec
            return (jnp.broadcast_to(jnp.max(cum), (_L,)),
                    b_acc + jnp.where(lt, 1, 0),
                    cb_acc + jnp.where(lt, h, 0))

        _, b_acc, cb_acc = lax.fori_loop(0, _NBUCKET // _L, scan_hist,
                                         (zeros, zeros, zeros))
        b_vec = jnp.broadcast_to(jnp.sum(b_acc), (_L,))
        cb_vec = jnp.broadcast_to(jnp.sum(cb_acc), (_L,))
        rrem = k_vec - cb_vec
        valid = k_vec > 0

        comp_v[...] = jnp.full((_L,), _SENTINEL, jnp.int32)

        def pass_b(i, off):
            m = m_v[pl.ds(i * _L, _L)]
            bkt = m >> _BUCKET_SHIFT
            out_v[pl.ds(i * _L, _L)] = jnp.where((bkt < b_vec) & valid, 1, 0)
            inb = bkt == b_vec
            pc = plsc.cumsum(jnp.where(inb, 1, 0))
            comb = ((m & _LOW_MASK) << 10) | (lane + i * _L)
            plsc.store_scatter(comp_v, [off + pc - 1], comb, mask=inb)
            return off + plsc.all_reduce_population_count(inb)

        lax.fori_loop(0, _NCHUNK, pass_b, zeros)

        srt = jnp.sort(comp_v[...])
        sel_val = jnp.where((lane < rrem) & valid, 1, 0)
        plsc.store_scatter(out_v, [srt & (_N - 1)], sel_val,
                           mask=srt != _SENTINEL)
        pltpu.sync_copy(out_v, out_hbm.at[wid])


@functools.partial(jax.jit, static_argnums=())
def _run_sc(ratio_rep, rand):
    mesh = plsc.VectorSubcoreMesh(core_axis_name="c", subcore_axis_name="s")
    fn = pl.kernel(
        _sc_body,
        out_type=jax.ShapeDtypeStruct((_ROWS, _N), jnp.int32),
        mesh=mesh,
        scratch_types=[
            pltpu.VMEM((_N,), jnp.float32),
            pltpu.VMEM((_N,), jnp.int32),
            pltpu.VMEM((_L,), jnp.float32),
            pltpu.VMEM((_NBUCKET,), jnp.int32),
            pltpu.VMEM((_L,), jnp.int32),
            pltpu.VMEM((_N,), jnp.int32),
            pltpu.SemaphoreType.DMA,
            pltpu.SemaphoreType.DMA,
        ],
        compiler_params=pltpu.CompilerParams(needs_layout_passes=False),
    )
    return fn(ratio_rep, rand)


def kernel(mask_frame, mask_ratio):
    B, T, C, H, W = mask_frame.shape
    h = H // _PATCH
    w = W // _PATCH
    rand = jax.random.uniform(jax.random.key(42), (_ROWS, _N),
                              dtype=jnp.float32)
    ratio_rep = jnp.broadcast_to(mask_ratio.astype(jnp.float32), (_L,))
    out = _run_sc(ratio_rep, rand)
    return out.astype(jnp.bool_).reshape(B, T, h, w)


# speedup vs baseline: 1.0338x; 1.0016x over previous
import functools

import jax
import jax.numpy as jnp
from jax import lax
from jax.experimental import pallas as pl
from jax.experimental.pallas import tpu as pltpu
from jax.experimental.pallas import tpu_sc as plsc

_PATCH = 16
_ROWS = 20
_N = 1024
_L = 16
_NCHUNK = _N // _L
_KEY_BITS = 23
_BUCKET_SHIFT = 15
_LOW_MASK = (1 << _BUCKET_SHIFT) - 1
_NBUCKET = 1 << (_KEY_BITS - _BUCKET_SHIFT)
_SENTINEL = 0x7FFFFFFF


def _sc_body(ratio_hbm, rand_hbm, out_hbm, row_v, m_v, ratio_v, hist_v,
             comp_v, out_v, sem_row, sem_ratio):
    wid = lax.axis_index("s") * 2 + lax.axis_index("c")

    @pl.when(wid < _ROWS)
    def _():
        cp_row = pltpu.make_async_copy(rand_hbm.at[wid], row_v, sem_row)
        cp_row.start()
        cp_ratio = pltpu.make_async_copy(ratio_hbm, ratio_v, sem_ratio)
        cp_ratio.start()
        lane = lax.iota(jnp.int32, _L)
        ones = jnp.ones((_L,), jnp.int32)
        zeros = jnp.zeros((_L,), jnp.int32)

        def zero_body(i, carry):
            hist_v[pl.ds(i * _L, _L)] = zeros
            return carry

        lax.fori_loop(0, _NBUCKET // _L, zero_body, 0)
        cp_row.wait()
        cp_ratio.wait()

        def pass_a(i, carry):
            x = row_v[pl.ds(i * _L, _L)]
            m = (x * float(1 << _KEY_BITS)).astype(jnp.int32)
            m_v[pl.ds(i * _L, _L)] = m
            plsc.addupdate_scatter(hist_v, [m >> _BUCKET_SHIFT], ones)
            return carry

        lax.fori_loop(0, _NCHUNK, pass_a, 0, unroll=2)

        ratio = ratio_v[...]
        k_vec = (ratio * float(_N)).astype(jnp.int32)

        def scan_hist(i, carry):
            run, b_acc, cb_acc = carry
            h = hist_v[pl.ds(i * _L, _L)]
            cum = plsc.cumsum(h) + run
            lt = cum < k_vec
            return (jnp.broadcast_to(jnp.max(cum), (_L,)),
                    b_acc + jnp.where(lt, 1, 0),
                    cb_acc + jnp.where(lt, h, 0))

        _, b_acc, cb_acc = lax.fori_loop(0, _NBUCKET // _L, scan_hist,
                                         (zeros, zeros, zeros))
        b_vec = jnp.broadcast_to(jnp.sum(b_acc), (_L,))
        cb_vec = jnp.broadcast_to(jnp.sum(cb_acc), (_L,))
        rrem = k_vec - cb_vec
        valid = k_vec > 0

        comp_v[...] = jnp.full((_L,), _SENTINEL, jnp.int32)

        def pass_b(i, off):
            m = m_v[pl.ds(i * _L, _L)]
            bkt = m >> _BUCKET_SHIFT
            out_v[pl.ds(i * _L, _L)] = jnp.where((bkt < b_vec) & valid, 1, 0)
            inb = bkt == b_vec
            pc = plsc.cumsum(jnp.where(inb, 1, 0))
            comb = ((m & _LOW_MASK) << 10) | (lane + i * _L)
            plsc.store_scatter(comp_v, [off + pc - 1], comb, mask=inb)
            return off + plsc.all_reduce_population_count(inb)

        lax.fori_loop(0, _NCHUNK, pass_b, zeros, unroll=2)

        srt = jnp.sort(comp_v[...])
        sel_val = jnp.where((lane < rrem) & valid, 1, 0)
        plsc.store_scatter(out_v, [srt & (_N - 1)], sel_val,
                           mask=srt != _SENTINEL)
        pltpu.sync_copy(out_v, out_hbm.at[wid])


@functools.partial(jax.jit, static_argnums=())
def _run_sc(ratio_rep, rand):
    mesh = plsc.VectorSubcoreMesh(core_axis_name="c", subcore_axis_name="s")
    fn = pl.kernel(
        _sc_body,
        out_type=jax.ShapeDtypeStruct((_ROWS, _N), jnp.int32),
        mesh=mesh,
        scratch_types=[
            pltpu.VMEM((_N,), jnp.float32),
            pltpu.VMEM((_N,), jnp.int32),
            pltpu.VMEM((_L,), jnp.float32),
            pltpu.VMEM((_NBUCKET,), jnp.int32),
            pltpu.VMEM((_L,), jnp.int32),
            pltpu.VMEM((_N,), jnp.int32),
            pltpu.SemaphoreType.DMA,
            pltpu.SemaphoreType.DMA,
        ],
        compiler_params=pltpu.CompilerParams(needs_layout_passes=False),
    )
    return fn(ratio_rep, rand)


def kernel(mask_frame, mask_ratio):
    B, T, C, H, W = mask_frame.shape
    h = H // _PATCH
    w = W // _PATCH
    rand = jax.random.uniform(jax.random.key(42), (_ROWS, _N),
                              dtype=jnp.float32)
    ratio_rep = jnp.broadcast_to(mask_ratio.astype(jnp.float32), (_L,))
    out = _run_sc(ratio_rep, rand)
    return out.astype(jnp.bool_).reshape(B, T, h, w)
